# baseline (device time: 18008 ns/iter reference)
import jax
import jax.numpy as jnp
from jax import lax
from jax.experimental import pallas as pl
from jax.experimental.pallas import tpu as pltpu

Z = 4
T = 512
D = 512
V_SHARD = 4096
TB = T // 4
C = 8
TBC = TB // C


def kernel(ids, E):
    my_x = lax.axis_index("x")
    my_y = lax.axis_index("y")
    my_z = lax.axis_index("z")
    p = 2 * my_x + my_y

    tok = lax.dynamic_slice_in_dim(ids, p * TB, TB)
    local = tok - my_z * V_SHARD
    valid = (local >= 0) & (local < V_SHARD)
    safe = jnp.clip(local, 0, V_SHARD - 1)
    partial = jnp.where(valid[:, None], jnp.take(E, safe, axis=0), 0.0)
    partial = partial.astype(jnp.bfloat16)

    def body(pb_ref, out_ref, exz_ref, sbf_ref, rbf_ref,
             s1_sems, r1_sems, s3_sems, r3_sems):
        x = lax.axis_index("x")
        y = lax.axis_index("y")
        z = lax.axis_index("z")
        myp = 2 * x + y
        xy_peers = [(1 - x, y, z), (x, 1 - y, z), (1 - x, 1 - y, z)]
        z_peers = [(x, y, (z + d) % Z) for d in (1, 2, 3)]

        def blk(c):
            return pl.ds(c * TBC, TBC)

        barrier_sem = pltpu.get_barrier_semaphore()
        for dev in z_peers + xy_peers:
            pl.semaphore_signal(
                barrier_sem, inc=1,
                device_id=dev, device_id_type=pl.DeviceIdType.MESH,
            )
        pl.semaphore_wait(barrier_sem, 6)

        rd1 = []
        for d in (1, 2, 3):
            for c in range(C):
                rdma = pltpu.make_async_remote_copy(
                    src_ref=pb_ref.at[blk(c), :],
                    dst_ref=exz_ref.at[3 - d, blk(c), :],
                    send_sem=s1_sems.at[d - 1, c],
                    recv_sem=r1_sems.at[3 - d, c],
                    device_id=z_peers[d - 1],
                    device_id_type=pl.DeviceIdType.MESH,
                )
                rdma.start()
                rd1.append(rdma)

        sends = []
        for c in range(C):
            for s in range(3):
                recv = pltpu.make_async_remote_copy(
                    src_ref=pb_ref.at[blk(c), :],
                    dst_ref=exz_ref.at[s, blk(c), :],
                    send_sem=s1_sems.at[s, c],
                    recv_sem=r1_sems.at[s, c],
                    device_id=z_peers[s],
                    device_id_type=pl.DeviceIdType.MESH,
                )
                recv.wait_recv()
            sbf_ref[blk(c), :] = (
                pb_ref[blk(c), :]
                + exz_ref[0, blk(c), :]
                + exz_ref[1, blk(c), :]
                + exz_ref[2, blk(c), :]
            )
            for q in xy_peers:
                qp = 2 * q[0] + q[1]
                rdma = pltpu.make_async_remote_copy(
                    src_ref=sbf_ref.at[blk(c), :],
                    dst_ref=rbf_ref.at[myp, blk(c), :],
                    send_sem=s3_sems.at[qp, c],
                    recv_sem=r3_sems.at[myp, c],
                    device_id=q,
                    device_id_type=pl.DeviceIdType.MESH,
                )
                rdma.start()
                sends.append(rdma)
            out_ref[pl.ds(myp * TB + c * TBC, TBC), :] = (
                sbf_ref[blk(c), :].astype(jnp.float32)
            )

        for q in xy_peers:
            qp = 2 * q[0] + q[1]
            for c in range(C):
                recv = pltpu.make_async_remote_copy(
                    src_ref=sbf_ref.at[blk(c), :],
                    dst_ref=rbf_ref.at[qp, blk(c), :],
                    send_sem=s3_sems.at[qp, c],
                    recv_sem=r3_sems.at[qp, c],
                    device_id=q,
                    device_id_type=pl.DeviceIdType.MESH,
                )
                recv.wait_recv()
            out_ref[pl.ds(qp * TB, TB), :] = rbf_ref[qp].astype(jnp.float32)
        for rdma in rd1 + sends:
            rdma.wait_send()

    return pl.pallas_call(
        body,
        out_shape=jax.ShapeDtypeStruct((T, D), jnp.float32),
        in_specs=[pl.BlockSpec(memory_space=pltpu.VMEM)],
        out_specs=pl.BlockSpec(memory_space=pltpu.VMEM),
        scratch_shapes=[
            pltpu.VMEM((3, TB, D), jnp.bfloat16),
            pltpu.VMEM((TB, D), jnp.bfloat16),
            pltpu.VMEM((4, TB, D), jnp.bfloat16),
            pltpu.SemaphoreType.DMA((3, C)),
            pltpu.SemaphoreType.DMA((3, C)),
            pltpu.SemaphoreType.DMA((4, C)),
            pltpu.SemaphoreType.DMA((4, C)),
        ],
        compiler_params=pltpu.CompilerParams(collective_id=0),
    )(partial)


# device time: 17646 ns/iter; 1.0205x vs baseline; 1.0205x over previous
import jax
import jax.numpy as jnp
from jax import lax
from jax.experimental import pallas as pl
from jax.experimental.pallas import tpu as pltpu

Z = 4
T = 512
D = 512
V_SHARD = 4096
TB = T // 4
C = 8
TBC = TB // C


def kernel(ids, E):
    my_x = lax.axis_index("x")
    my_y = lax.axis_index("y")
    my_z = lax.axis_index("z")
    p = 2 * my_x + my_y

    tok = lax.dynamic_slice_in_dim(ids, p * TB, TB)
    local = tok - my_z * V_SHARD
    valid = (local >= 0) & (local < V_SHARD)
    safe = jnp.clip(local, 0, V_SHARD - 1)
    partial = jnp.where(valid[:, None], jnp.take(E, safe, axis=0), 0.0)
    partial = partial.astype(jnp.bfloat16)

    def body(pb_ref, out_ref, ex1_ref, ex2_ref, sbf_ref, rbf_ref,
             s1_sems, r1_sems, s2_sems, r2_sems, s3_sems, r3_sems):
        x = lax.axis_index("x")
        y = lax.axis_index("y")
        z = lax.axis_index("z")
        myp = 2 * x + y
        xy_peers = [(1 - x, y, z), (x, 1 - y, z), (1 - x, 1 - y, z)]
        z_peers = [(x, y, z ^ 1), (x, y, z ^ 2)]

        def blk(c):
            return pl.ds(c * TBC, TBC)

        def out_blk(bp, c):
            return pl.ds(bp * TB + c * TBC, TBC)

        barrier_sem = pltpu.get_barrier_semaphore()
        for d in z_peers + xy_peers:
            pl.semaphore_signal(
                barrier_sem, inc=1,
                device_id=d, device_id_type=pl.DeviceIdType.MESH,
            )
        pl.semaphore_wait(barrier_sem, 5)

        rd1 = []
        for c in range(C):
            rdma = pltpu.make_async_remote_copy(
                src_ref=pb_ref.at[blk(c), :],
                dst_ref=ex1_ref.at[blk(c), :],
                send_sem=s1_sems.at[c],
                recv_sem=r1_sems.at[c],
                device_id=z_peers[0],
                device_id_type=pl.DeviceIdType.MESH,
            )
            rdma.start()
            rd1.append(rdma)

        rd2 = []
        for c in range(C):
            rd1[c].wait_recv()
            sbf_ref[blk(c), :] = pb_ref[blk(c), :] + ex1_ref[blk(c), :]
            rdma = pltpu.make_async_remote_copy(
                src_ref=sbf_ref.at[blk(c), :],
                dst_ref=ex2_ref.at[blk(c), :],
                send_sem=s2_sems.at[c],
                recv_sem=r2_sems.at[c],
                device_id=z_peers[1],
                device_id_type=pl.DeviceIdType.MESH,
            )
            rdma.start()
            rd2.append(rdma)

        sends = []
        for c in range(C):
            rd2[c].wait()
            sbf_ref[blk(c), :] += ex2_ref[blk(c), :]
            for q in xy_peers:
                qp = 2 * q[0] + q[1]
                rdma = pltpu.make_async_remote_copy(
                    src_ref=sbf_ref.at[blk(c), :],
                    dst_ref=rbf_ref.at[myp, blk(c), :],
                    send_sem=s3_sems.at[qp, c],
                    recv_sem=r3_sems.at[myp, c],
                    device_id=q,
                    device_id_type=pl.DeviceIdType.MESH,
                )
                rdma.start()
                sends.append(rdma)
            out_ref[out_blk(myp, c), :] = sbf_ref[blk(c), :].astype(jnp.float32)

        for q in xy_peers:
            qp = 2 * q[0] + q[1]
            for c in range(C):
                recv = pltpu.make_async_remote_copy(
                    src_ref=sbf_ref.at[blk(c), :],
                    dst_ref=rbf_ref.at[qp, blk(c), :],
                    send_sem=s3_sems.at[qp, c],
                    recv_sem=r3_sems.at[qp, c],
                    device_id=q,
                    device_id_type=pl.DeviceIdType.MESH,
                )
                recv.wait_recv()
                out_ref[out_blk(qp, c), :] = (
                    rbf_ref[qp, blk(c), :].astype(jnp.float32)
                )
        for rdma in rd1 + sends:
            rdma.wait_send()

    return pl.pallas_call(
        body,
        out_shape=jax.ShapeDtypeStruct((T, D), jnp.float32),
        in_specs=[pl.BlockSpec(memory_space=pltpu.VMEM)],
        out_specs=pl.BlockSpec(memory_space=pltpu.VMEM),
        scratch_shapes=[
            pltpu.VMEM((TB, D), jnp.bfloat16),
            pltpu.VMEM((TB, D), jnp.bfloat16),
            pltpu.VMEM((TB, D), jnp.bfloat16),
            pltpu.VMEM((4, TB, D), jnp.bfloat16),
            pltpu.SemaphoreType.DMA((C,)),
            pltpu.SemaphoreType.DMA((C,)),
            pltpu.SemaphoreType.DMA((C,)),
            pltpu.SemaphoreType.DMA((C,)),
            pltpu.SemaphoreType.DMA((4, C)),
            pltpu.SemaphoreType.DMA((4, C)),
        ],
        compiler_params=pltpu.CompilerParams(collective_id=0),
    )(partial)
